# baseline (device time: 211665 ns/iter reference)
import jax
import jax.numpy as jnp
from jax import lax
from jax.experimental import pallas as pl
from jax.experimental.pallas import tpu as pltpu

N_DEV = 16
N_TOK = 1024
D_IN = 512
D_OUT = 1024
E_PER = 4
CHUNK = N_TOK // N_DEV


def kernel(x, router_W, route_idx, expert_W):
    del router_W

    def body(x_ref, route_ref, ew_ref, out_ref,
             partial_ref, comm_ref, send_sems, recv_sems, credit_sem):
        my = lax.axis_index("i")
        left = jnp.mod(my - 1, N_DEV)
        right = jnp.mod(my + 1, N_DEV)

        barrier = pltpu.get_barrier_semaphore()
        for nbr in (left, right):
            pl.semaphore_signal(barrier, inc=1, device_id=(nbr,),
                                device_id_type=pl.DeviceIdType.MESH)
        pl.semaphore_wait(barrier, 2)

        xb = x_ref[:, :].astype(jnp.bfloat16)
        route = route_ref[:, :]
        acc = jnp.zeros((N_TOK, D_OUT), jnp.float32)
        for e in range(E_PER):
            gid = my * E_PER + e
            xm = jnp.where(route == gid, xb, jnp.bfloat16(0.0))
            acc = acc + jnp.dot(xm, ew_ref[e, :, :].astype(jnp.bfloat16),
                                preferred_element_type=jnp.float32)
        partial_ref[:, :] = acc.astype(jnp.bfloat16)

        comm_ref[0, :, :] = partial_ref[pl.ds(my * CHUNK, CHUNK), :]
        for s in range(2 * (N_DEV - 1)):
            send_slot = s % 2
            recv_slot = (s + 1) % 2
            if s >= 1:
                pl.semaphore_signal(credit_sem, inc=1, device_id=(left,),
                                    device_id_type=pl.DeviceIdType.MESH)
                pl.semaphore_wait(credit_sem, 1)
            rdma = pltpu.make_async_remote_copy(
                src_ref=comm_ref.at[send_slot],
                dst_ref=comm_ref.at[recv_slot],
                send_sem=send_sems.at[send_slot],
                recv_sem=recv_sems.at[recv_slot],
                device_id=(right,),
                device_id_type=pl.DeviceIdType.MESH,
            )
            rdma.start()
            rdma.wait()
            if s < N_DEV - 1:
                c = jnp.mod(my - s - 1, N_DEV)
                comm_ref[recv_slot, :, :] = (
                    comm_ref[recv_slot, :, :]
                    + partial_ref[pl.ds(c * CHUNK, CHUNK), :]
                )
                if s == N_DEV - 2:
                    out_ref[pl.ds(jnp.mod(my + 1, N_DEV) * CHUNK, CHUNK), :] = (
                        comm_ref[recv_slot, :, :].astype(jnp.float32))
            else:
                c = jnp.mod(my + (N_DEV - 1) - s, N_DEV)
                out_ref[pl.ds(c * CHUNK, CHUNK), :] = (
                    comm_ref[recv_slot, :, :].astype(jnp.float32))

    return pl.pallas_call(
        body,
        out_shape=jax.ShapeDtypeStruct((N_TOK, D_OUT), jnp.float32),
        in_specs=[
            pl.BlockSpec(memory_space=pltpu.VMEM),
            pl.BlockSpec(memory_space=pltpu.VMEM),
            pl.BlockSpec(memory_space=pltpu.VMEM),
        ],
        out_specs=pl.BlockSpec(memory_space=pltpu.VMEM),
        scratch_shapes=[
            pltpu.VMEM((N_TOK, D_OUT), jnp.bfloat16),
            pltpu.VMEM((2, CHUNK, D_OUT), jnp.bfloat16),
            pltpu.SemaphoreType.DMA((2,)),
            pltpu.SemaphoreType.DMA((2,)),
            pltpu.SemaphoreType.REGULAR,
        ],
        compiler_params=pltpu.CompilerParams(collective_id=0),
    )(x, route_idx, expert_W)


# device time: 75510 ns/iter; 2.8031x vs baseline; 2.8031x over previous
import jax
import jax.numpy as jnp
from jax import lax
from jax.experimental import pallas as pl
from jax.experimental.pallas import tpu as pltpu

N_DEV = 16
N_TOK = 1024
D_IN = 512
D_OUT = 1024
E_PER = 4

_RS_STAGES = ((0, 512), (2, 256), (1, 128), (3, 64))
_AG_STAGES = ((3, 64), (1, 128), (2, 256), (0, 512))


def kernel(x, router_W, route_idx, expert_W):
    del router_W

    def body(x_ref, route_ref, ew_ref, out_ref,
             partial_ref, rs0, rs1, rs2, rs3, send_sems, recv_sems):
        my = lax.axis_index("i")
        rs_bufs = [rs0, rs1, rs2, rs3]

        barrier = pltpu.get_barrier_semaphore()
        for bit in range(4):
            partner = jnp.bitwise_xor(my, 1 << bit)
            pl.semaphore_signal(barrier, inc=1, device_id=(partner,),
                                device_id_type=pl.DeviceIdType.MESH)
        pl.semaphore_wait(barrier, 4)

        xb = x_ref[:, :].astype(jnp.bfloat16)
        route = route_ref[:, :]
        acc = jnp.zeros((N_TOK, D_OUT), jnp.float32)
        for e in range(E_PER):
            gid = my * E_PER + e
            xm = jnp.where(route == gid, xb, jnp.bfloat16(0.0))
            acc = acc + jnp.dot(xm, ew_ref[e, :, :].astype(jnp.bfloat16),
                                preferred_element_type=jnp.float32)
        partial_ref[:, :] = acc.astype(jnp.bfloat16)

        pending = []

        lo = jnp.int32(0)
        for k, (bit, half) in enumerate(_RS_STAGES):
            b = jnp.bitwise_and(lax.shift_right_logical(my, bit), 1)
            partner = jnp.bitwise_xor(my, 1 << bit)
            keep_lo = lo + b * half
            send_lo = lo + (1 - b) * half
            rdma = pltpu.make_async_remote_copy(
                src_ref=partial_ref.at[pl.ds(send_lo, half), :],
                dst_ref=rs_bufs[k],
                send_sem=send_sems.at[k],
                recv_sem=recv_sems.at[k],
                device_id=(partner,),
                device_id_type=pl.DeviceIdType.MESH,
            )
            rdma.start()
            rdma.wait_recv()
            partial_ref[pl.ds(keep_lo, half), :] = (
                partial_ref[pl.ds(keep_lo, half), :] + rs_bufs[k][:, :]
            )
            pending.append(rdma)
            lo = keep_lo

        for j, (bit, sz) in enumerate(_AG_STAGES):
            b = jnp.bitwise_and(lax.shift_right_logical(my, bit), 1)
            partner = jnp.bitwise_xor(my, 1 << bit)
            rdma = pltpu.make_async_remote_copy(
                src_ref=partial_ref.at[pl.ds(lo, sz), :],
                dst_ref=partial_ref.at[pl.ds(lo, sz), :],
                send_sem=send_sems.at[4 + j],
                recv_sem=recv_sems.at[4 + j],
                device_id=(partner,),
                device_id_type=pl.DeviceIdType.MESH,
            )
            rdma.start()
            rdma.wait_recv()
            pending.append(rdma)
            lo = lo - b * sz

        out_ref[:, :] = partial_ref[:, :].astype(jnp.float32)

        for rdma in pending:
            rdma.wait_send()

    return pl.pallas_call(
        body,
        out_shape=jax.ShapeDtypeStruct((N_TOK, D_OUT), jnp.float32),
        in_specs=[
            pl.BlockSpec(memory_space=pltpu.VMEM),
            pl.BlockSpec(memory_space=pltpu.VMEM),
            pl.BlockSpec(memory_space=pltpu.VMEM),
        ],
        out_specs=pl.BlockSpec(memory_space=pltpu.VMEM),
        scratch_shapes=[
            pltpu.VMEM((N_TOK, D_OUT), jnp.bfloat16),
            pltpu.VMEM((512, D_OUT), jnp.bfloat16),
            pltpu.VMEM((256, D_OUT), jnp.bfloat16),
            pltpu.VMEM((128, D_OUT), jnp.bfloat16),
            pltpu.VMEM((64, D_OUT), jnp.bfloat16),
            pltpu.SemaphoreType.DMA((8,)),
            pltpu.SemaphoreType.DMA((8,)),
        ],
        compiler_params=pltpu.CompilerParams(collective_id=0),
    )(x, route_idx, expert_W)
